# baseline (speedup 1.0000x reference)
import functools

import jax
import jax.numpy as jnp
from jax import lax
from jax.experimental import pallas as pl
from jax.experimental.pallas import tpu as pltpu
from jax.experimental.pallas import tpu_sc as plsc

S, B, H, FF, E = 2048, 1, 1024, 2048, 8
TOP_P = 0.3
SB = S * B
K = 3
A = SB * K
BS_G = 128
NBLK = A // BS_G + E
NPAD = NBLK * BS_G
FB = 512
NW = 32


def _sc_gather(table, idx, n_rows, chunk):
    per_w = n_rows // NW
    nch = per_w // chunk
    h = table.shape[1]
    mesh = plsc.VectorSubcoreMesh(core_axis_name="c", subcore_axis_name="s")

    @functools.partial(
        pl.kernel,
        out_type=jax.ShapeDtypeStruct((n_rows, h), jnp.float32),
        mesh=mesh,
        scratch_types=[
            pltpu.VMEM((chunk,), jnp.int32),
            pltpu.VMEM((chunk, h), jnp.float32),
            pltpu.SemaphoreType.DMA,
        ],
    )
    def k(table_hbm, idx_hbm, out_hbm, idx_v, rows_v, sem):
        wid = lax.axis_index("s") * 2 + lax.axis_index("c")
        base = pl.multiple_of(wid * per_w, chunk)

        def body(ci, carry):
            off = pl.multiple_of(base + ci * chunk, 8)
            pltpu.sync_copy(idx_hbm.at[pl.ds(off, chunk)], idx_v)
            pltpu.async_copy(table_hbm.at[idx_v], rows_v, sem).wait()
            pltpu.sync_copy(rows_v, out_hbm.at[pl.ds(off, chunk)])
            return carry

        lax.fori_loop(0, nch, body, 0)

    return k(table, idx)


def _grouped_mlp(xg, gate_w, up_w, down_w, block_expert):
    grid_spec = pltpu.PrefetchScalarGridSpec(
        num_scalar_prefetch=1,
        grid=(NBLK, FF // FB),
        in_specs=[
            pl.BlockSpec((BS_G, H), lambda b, f, be: (b, 0)),
            pl.BlockSpec((1, FB, H), lambda b, f, be: (be[b], f, 0)),
            pl.BlockSpec((1, FB, H), lambda b, f, be: (be[b], f, 0)),
            pl.BlockSpec((1, H, FB), lambda b, f, be: (be[b], 0, f)),
        ],
        out_specs=pl.BlockSpec((BS_G, H), lambda b, f, be: (b, 0)),
    )

    def body(be_ref, x_ref, g_ref, u_ref, d_ref, o_ref):
        f = pl.program_id(1)
        x = x_ref[...]
        g = lax.dot_general(x, g_ref[0], (((1,), (1,)), ((), ())),
                            preferred_element_type=jnp.float32)
        u = lax.dot_general(x, u_ref[0], (((1,), (1,)), ((), ())),
                            preferred_element_type=jnp.float32)
        hmid = (g * jax.nn.sigmoid(g)) * u
        y = lax.dot_general(hmid, d_ref[0], (((1,), (1,)), ((), ())),
                            preferred_element_type=jnp.float32)

        @pl.when(f == 0)
        def _():
            o_ref[...] = jnp.zeros_like(o_ref)

        o_ref[...] += y

    return pl.pallas_call(
        body,
        grid_spec=grid_spec,
        out_shape=jax.ShapeDtypeStruct((NPAD, H), jnp.float32),
        compiler_params=pltpu.CompilerParams(
            dimension_semantics=("arbitrary", "arbitrary"),
        ),
    )(block_expert, xg, gate_w, up_w, down_w)


def _combine(yg3, w3):
    bsr = 256

    def body(y_ref, w_ref, o_ref):
        y = y_ref[...]
        w = w_ref[...]
        o_ref[...] = (y[0] * w[0][:, None] + y[1] * w[1][:, None]
                      + y[2] * w[2][:, None])

    return pl.pallas_call(
        body,
        grid=(SB // bsr,),
        in_specs=[
            pl.BlockSpec((K, bsr, H), lambda i: (0, i, 0)),
            pl.BlockSpec((K, bsr), lambda i: (0, i)),
        ],
        out_specs=pl.BlockSpec((bsr, H), lambda i: (i, 0)),
        out_shape=jax.ShapeDtypeStruct((SB, H), jnp.float32),
    )(yg3, w3)


def kernel(hidden_states, router_w, gate_w, up_w, down_w):
    x = hidden_states.reshape(SB, H)

    logits = x @ router_w.T
    probs = jax.nn.softmax(logits, axis=-1)
    p3, i3 = lax.top_k(probs, K)
    keep1 = p3[:, 0] <= TOP_P
    keep2 = (p3[:, 0] + p3[:, 1]) <= TOP_P
    w3 = jnp.stack([p3[:, 0],
                    jnp.where(keep1, p3[:, 1], 0.0),
                    jnp.where(keep2, p3[:, 2], 0.0)], axis=0)

    ef = i3.reshape(-1).astype(jnp.int32)
    counts = jnp.sum((ef[None, :] == jnp.arange(E)[:, None]).astype(jnp.int32),
                     axis=1)
    nb = (counts + BS_G - 1) // BS_G
    cnb = jnp.cumsum(nb)
    base = (cnb - nb) * BS_G
    order = jnp.argsort(ef, stable=True)
    sef = ef[order]
    gstart = jnp.cumsum(counts) - counts
    slot = base[sef] + (jnp.arange(A, dtype=jnp.int32) - gstart[sef])
    tok = jnp.zeros((NPAD,), jnp.int32).at[slot].set(
        (order // K).astype(jnp.int32))
    posj = jnp.zeros((A,), jnp.int32).at[order].set(slot.astype(jnp.int32))
    pos_sm = posj.reshape(SB, K).T.reshape(A)
    block_expert = jnp.minimum(
        jnp.searchsorted(cnb, jnp.arange(NBLK), side='right'),
        E - 1).astype(jnp.int32)

    xg = _sc_gather(x, tok, NPAD, 56)
    yg = _grouped_mlp(xg, gate_w, up_w, down_w, block_expert)
    yg3 = _sc_gather(yg, pos_sm, A, 48).reshape(K, SB, H)
    out = _combine(yg3, w3)
    return out.reshape(S, B, H)


# speedup vs baseline: 1.4214x; 1.4214x over previous
import functools

import jax
import jax.numpy as jnp
from jax import lax
from jax.experimental import pallas as pl
from jax.experimental.pallas import tpu as pltpu
from jax.experimental.pallas import tpu_sc as plsc

S, B, H, FF, E = 2048, 1, 1024, 2048, 8
TOP_P = 0.3
SB = S * B
K = 3
A = SB * K
BS_G = 128
NBLK = A // BS_G + E
NPAD = NBLK * BS_G
FB = 512
NW = 32


def _sc_gather(table, idx, n_rows, chunk):
    per_w = n_rows // NW
    nch = per_w // chunk
    h = table.shape[1]
    mesh = plsc.VectorSubcoreMesh(core_axis_name="c", subcore_axis_name="s")

    @functools.partial(
        pl.kernel,
        out_type=jax.ShapeDtypeStruct((n_rows, h), jnp.float32),
        mesh=mesh,
        scratch_types=[
            pltpu.VMEM((chunk,), jnp.int32),
            pltpu.VMEM((chunk, h), jnp.float32),
            pltpu.SemaphoreType.DMA,
        ],
    )
    def k(table_hbm, idx_hbm, out_hbm, idx_v, rows_v, sem):
        wid = lax.axis_index("s") * 2 + lax.axis_index("c")
        base = pl.multiple_of(wid * per_w, chunk)

        def body(ci, carry):
            off = pl.multiple_of(base + ci * chunk, 8)
            pltpu.sync_copy(idx_hbm.at[pl.ds(off, chunk)], idx_v)
            pltpu.async_copy(table_hbm.at[idx_v], rows_v, sem).wait()
            pltpu.sync_copy(rows_v, out_hbm.at[pl.ds(off, chunk)])
            return carry

        lax.fori_loop(0, nch, body, 0)

    return k(table, idx)


def _grouped_mlp(xg, gate_w, up_w, down_w, block_expert):
    grid_spec = pltpu.PrefetchScalarGridSpec(
        num_scalar_prefetch=1,
        grid=(NBLK,),
        in_specs=[
            pl.BlockSpec((BS_G, H), lambda b, be: (b, 0)),
            pl.BlockSpec((1, FF, H), lambda b, be: (be[b], 0, 0)),
            pl.BlockSpec((1, FF, H), lambda b, be: (be[b], 0, 0)),
            pl.BlockSpec((1, H, FF), lambda b, be: (be[b], 0, 0)),
        ],
        out_specs=pl.BlockSpec((BS_G, H), lambda b, be: (b, 0)),
    )

    def body(be_ref, x_ref, g_ref, u_ref, d_ref, o_ref):
        x = x_ref[...]
        g = lax.dot_general(x, g_ref[0], (((1,), (1,)), ((), ())),
                            preferred_element_type=jnp.float32)
        u = lax.dot_general(x, u_ref[0], (((1,), (1,)), ((), ())),
                            preferred_element_type=jnp.float32)
        hmid = (g * jax.nn.sigmoid(g)) * u
        o_ref[...] = lax.dot_general(hmid, d_ref[0], (((1,), (1,)), ((), ())),
                                     preferred_element_type=jnp.float32)

    return pl.pallas_call(
        body,
        grid_spec=grid_spec,
        out_shape=jax.ShapeDtypeStruct((NPAD, H), jnp.float32),
        compiler_params=pltpu.CompilerParams(
            dimension_semantics=("arbitrary",),
        ),
    )(block_expert, xg, gate_w, up_w, down_w)


def _combine(yg3, w3):
    bsr = 256

    def body(y_ref, w_ref, o_ref):
        y = y_ref[...]
        w = w_ref[...]
        o_ref[...] = (y[0] * w[0][:, None] + y[1] * w[1][:, None]
                      + y[2] * w[2][:, None])

    return pl.pallas_call(
        body,
        grid=(SB // bsr,),
        in_specs=[
            pl.BlockSpec((K, bsr, H), lambda i: (0, i, 0)),
            pl.BlockSpec((K, bsr), lambda i: (0, i)),
        ],
        out_specs=pl.BlockSpec((bsr, H), lambda i: (i, 0)),
        out_shape=jax.ShapeDtypeStruct((SB, H), jnp.float32),
    )(yg3, w3)


def kernel(hidden_states, router_w, gate_w, up_w, down_w):
    x = hidden_states.reshape(SB, H)

    logits = x @ router_w.T
    probs = jax.nn.softmax(logits, axis=-1)
    p3, i3 = lax.top_k(probs, K)
    keep1 = p3[:, 0] <= TOP_P
    keep2 = (p3[:, 0] + p3[:, 1]) <= TOP_P
    w3 = jnp.stack([p3[:, 0],
                    jnp.where(keep1, p3[:, 1], 0.0),
                    jnp.where(keep2, p3[:, 2], 0.0)], axis=0)

    ef = i3.reshape(-1).astype(jnp.int32)
    counts = jnp.sum((ef[None, :] == jnp.arange(E)[:, None]).astype(jnp.int32),
                     axis=1)
    nb = (counts + BS_G - 1) // BS_G
    cnb = jnp.cumsum(nb)
    base = (cnb - nb) * BS_G
    order = jnp.argsort(ef, stable=True)
    sef = ef[order]
    gstart = jnp.cumsum(counts) - counts
    slot = base[sef] + (jnp.arange(A, dtype=jnp.int32) - gstart[sef])
    tok = jnp.zeros((NPAD,), jnp.int32).at[slot].set(
        (order // K).astype(jnp.int32))
    posj = jnp.zeros((A,), jnp.int32).at[order].set(slot.astype(jnp.int32))
    pos_sm = posj.reshape(SB, K).T.reshape(A)
    block_expert = jnp.minimum(
        jnp.searchsorted(cnb, jnp.arange(NBLK), side='right'),
        E - 1).astype(jnp.int32)

    xg = _sc_gather(x, tok, NPAD, 56)
    yg = _grouped_mlp(xg, gate_w, up_w, down_w, block_expert)
    yg3 = _sc_gather(yg, pos_sm, A, 48).reshape(K, SB, H)
    out = _combine(yg3, w3)
    return out.reshape(S, B, H)


# speedup vs baseline: 1.4470x; 1.0180x over previous
import functools

import jax
import jax.numpy as jnp
from jax import lax
from jax.experimental import pallas as pl
from jax.experimental.pallas import tpu as pltpu
from jax.experimental.pallas import tpu_sc as plsc

S, B, H, FF, E = 2048, 1, 1024, 2048, 8
TOP_P = 0.3
SB = S * B
K = 3
A = SB * K
BS_G = 128
NBLK = A // BS_G + E
NPAD = NBLK * BS_G
FB = 512
NW = 32


def _sc_gather(table, idx, n_rows, chunk):
    per_w = n_rows // NW
    nch = per_w // chunk
    h = table.shape[1]
    mesh = plsc.VectorSubcoreMesh(core_axis_name="c", subcore_axis_name="s")

    @functools.partial(
        pl.kernel,
        out_type=jax.ShapeDtypeStruct((n_rows, h), jnp.float32),
        mesh=mesh,
        scratch_types=[
            pltpu.VMEM((chunk,), jnp.int32),
            pltpu.VMEM((chunk, h), jnp.float32),
            pltpu.SemaphoreType.DMA,
        ],
    )
    def k(table_hbm, idx_hbm, out_hbm, idx_v, rows_v, sem):
        wid = lax.axis_index("s") * 2 + lax.axis_index("c")
        base = pl.multiple_of(wid * per_w, chunk)

        def body(ci, carry):
            off = pl.multiple_of(base + ci * chunk, 8)
            pltpu.sync_copy(idx_hbm.at[pl.ds(off, chunk)], idx_v)
            pltpu.async_copy(table_hbm.at[idx_v], rows_v, sem).wait()
            pltpu.sync_copy(rows_v, out_hbm.at[pl.ds(off, chunk)])
            return carry

        lax.fori_loop(0, nch, body, 0)

    return k(table, idx)


def _grouped_mlp(xg, gate_w, up_w, down_w, block_expert):
    grid_spec = pltpu.PrefetchScalarGridSpec(
        num_scalar_prefetch=1,
        grid=(NBLK,),
        in_specs=[
            pl.BlockSpec((BS_G, H), lambda b, be: (b, 0)),
            pl.BlockSpec((1, FF, H), lambda b, be: (be[b], 0, 0)),
            pl.BlockSpec((1, FF, H), lambda b, be: (be[b], 0, 0)),
            pl.BlockSpec((1, H, FF), lambda b, be: (be[b], 0, 0)),
        ],
        out_specs=pl.BlockSpec((BS_G, H), lambda b, be: (b, 0)),
    )

    def body(be_ref, x_ref, g_ref, u_ref, d_ref, o_ref):
        x = x_ref[...]
        g = lax.dot_general(x, g_ref[0], (((1,), (1,)), ((), ())),
                            preferred_element_type=jnp.float32)
        u = lax.dot_general(x, u_ref[0], (((1,), (1,)), ((), ())),
                            preferred_element_type=jnp.float32)
        hmid = (g * jax.nn.sigmoid(g)) * u
        o_ref[...] = lax.dot_general(hmid, d_ref[0], (((1,), (1,)), ((), ())),
                                     preferred_element_type=jnp.float32)

    return pl.pallas_call(
        body,
        grid_spec=grid_spec,
        out_shape=jax.ShapeDtypeStruct((NPAD, H), jnp.float32),
        compiler_params=pltpu.CompilerParams(
            dimension_semantics=("arbitrary",),
        ),
    )(block_expert, xg, gate_w, up_w, down_w)


def _combine(yg3, w3):
    bsr = 256

    def body(y_ref, w_ref, o_ref):
        y = y_ref[...]
        w = w_ref[...]
        o_ref[...] = (y[0] * w[0][:, None] + y[1] * w[1][:, None]
                      + y[2] * w[2][:, None])

    return pl.pallas_call(
        body,
        grid=(SB // bsr,),
        in_specs=[
            pl.BlockSpec((K, bsr, H), lambda i: (0, i, 0)),
            pl.BlockSpec((K, bsr), lambda i: (0, i)),
        ],
        out_specs=pl.BlockSpec((bsr, H), lambda i: (i, 0)),
        out_shape=jax.ShapeDtypeStruct((SB, H), jnp.float32),
    )(yg3, w3)


def kernel(hidden_states, router_w, gate_w, up_w, down_w):
    x = hidden_states.reshape(SB, H)

    logits = x @ router_w.T
    probs = jax.nn.softmax(logits, axis=-1)
    p3, i3 = lax.top_k(probs, K)
    keep1 = p3[:, 0] <= TOP_P
    keep2 = (p3[:, 0] + p3[:, 1]) <= TOP_P
    w3 = jnp.stack([p3[:, 0],
                    jnp.where(keep1, p3[:, 1], 0.0),
                    jnp.where(keep2, p3[:, 2], 0.0)], axis=0)

    ef = i3.reshape(-1).astype(jnp.int32)
    cb = 128
    nc = A // cb
    oh = (ef.reshape(nc, cb)[:, :, None]
          == jnp.arange(E, dtype=jnp.int32)[None, None, :])
    ohf = oh.astype(jnp.float32)
    tri = (jnp.arange(cb)[:, None] >= jnp.arange(cb)[None, :]).astype(
        jnp.float32)
    intra = jnp.einsum('rc,ncE->nrE', tri, ohf,
                       preferred_element_type=jnp.float32)
    chunk_tot = intra[:, cb - 1, :]
    carry = jnp.cumsum(chunk_tot, axis=0) - chunk_tot
    counts = chunk_tot.sum(axis=0).astype(jnp.int32)
    rank = (intra + carry[:, None, :]).astype(jnp.int32).reshape(A, E)
    rank = jnp.take_along_axis(rank, ef[:, None], axis=1)[:, 0] - 1
    nb = (counts + BS_G - 1) // BS_G
    cnb = jnp.cumsum(nb)
    base = (cnb - nb) * BS_G
    slot = base[ef] + rank
    tok = jnp.zeros((NPAD,), jnp.int32).at[slot].set(
        jnp.arange(A, dtype=jnp.int32) // K)
    pos_sm = slot.reshape(SB, K).T.reshape(A)
    block_expert = jnp.minimum(
        jnp.searchsorted(cnb, jnp.arange(NBLK), side='right'),
        E - 1).astype(jnp.int32)

    xg = _sc_gather(x, tok, NPAD, 56)
    yg = _grouped_mlp(xg, gate_w, up_w, down_w, block_expert)
    yg3 = _sc_gather(yg, pos_sm, A, 48).reshape(K, SB, H)
    out = _combine(yg3, w3)
    return out.reshape(S, B, H)


# speedup vs baseline: 1.4522x; 1.0036x over previous
import functools

import jax
import jax.numpy as jnp
from jax import lax
from jax.experimental import pallas as pl
from jax.experimental.pallas import tpu as pltpu
from jax.experimental.pallas import tpu_sc as plsc

S, B, H, FF, E = 2048, 1, 1024, 2048, 8
TOP_P = 0.3
SB = S * B
K = 3
A = SB * K
BS_G = 128
NBLK = A // BS_G + E
NPAD = NBLK * BS_G
FB = 512
NW = 32


def _sc_gather(table, idx, n_rows, chunk):
    per_w = n_rows // NW
    nch = per_w // chunk
    h = table.shape[1]
    mesh = plsc.VectorSubcoreMesh(core_axis_name="c", subcore_axis_name="s")

    @functools.partial(
        pl.kernel,
        out_type=jax.ShapeDtypeStruct((n_rows, h), jnp.float32),
        mesh=mesh,
        scratch_types=[
            pltpu.VMEM((per_w,), jnp.int32),
            pltpu.VMEM((chunk, h), jnp.float32),
            pltpu.VMEM((chunk, h), jnp.float32),
            pltpu.SemaphoreType.DMA,
            pltpu.SemaphoreType.DMA,
        ],
    )
    def k(table_hbm, idx_hbm, out_hbm, idx_v, rows0, rows1, sem0, sem1):
        wid = lax.axis_index("s") * 2 + lax.axis_index("c")
        base = pl.multiple_of(wid * per_w, chunk)
        pltpu.sync_copy(idx_hbm.at[pl.ds(base, per_w)], idx_v)
        bufs = (rows0, rows1)
        sems = (sem0, sem1)
        copies = [
            pltpu.async_copy(
                table_hbm.at[idx_v.at[pl.ds(ci * chunk, chunk)]],
                bufs[ci % 2], sems[ci % 2])
            for ci in range(min(2, nch))
        ]
        for ci in range(nch):
            copies[ci].wait()
            pltpu.sync_copy(
                bufs[ci % 2],
                out_hbm.at[pl.ds(pl.multiple_of(base + ci * chunk, 8), chunk)])
            if ci + 2 < nch:
                copies.append(pltpu.async_copy(
                    table_hbm.at[idx_v.at[pl.ds((ci + 2) * chunk, chunk)]],
                    bufs[ci % 2], sems[ci % 2]))

    return k(table, idx)


def _grouped_mlp(xg, gate_w, up_w, down_w, block_expert):
    grid_spec = pltpu.PrefetchScalarGridSpec(
        num_scalar_prefetch=1,
        grid=(NBLK,),
        in_specs=[
            pl.BlockSpec((BS_G, H), lambda b, be: (b, 0)),
            pl.BlockSpec((1, FF, H), lambda b, be: (be[b], 0, 0)),
            pl.BlockSpec((1, FF, H), lambda b, be: (be[b], 0, 0)),
            pl.BlockSpec((1, H, FF), lambda b, be: (be[b], 0, 0)),
        ],
        out_specs=pl.BlockSpec((BS_G, H), lambda b, be: (b, 0)),
    )

    def body(be_ref, x_ref, g_ref, u_ref, d_ref, o_ref):
        x = x_ref[...]
        g = lax.dot_general(x, g_ref[0], (((1,), (1,)), ((), ())),
                            preferred_element_type=jnp.float32)
        u = lax.dot_general(x, u_ref[0], (((1,), (1,)), ((), ())),
                            preferred_element_type=jnp.float32)
        hmid = (g * jax.nn.sigmoid(g)) * u
        o_ref[...] = lax.dot_general(hmid, d_ref[0], (((1,), (1,)), ((), ())),
                                     preferred_element_type=jnp.float32)

    return pl.pallas_call(
        body,
        grid_spec=grid_spec,
        out_shape=jax.ShapeDtypeStruct((NPAD, H), jnp.float32),
        compiler_params=pltpu.CompilerParams(
            dimension_semantics=("arbitrary",),
        ),
    )(block_expert, xg, gate_w, up_w, down_w)


def _combine(yg3, w3):
    bsr = 256

    def body(y_ref, w_ref, o_ref):
        y = y_ref[...]
        w = w_ref[...]
        o_ref[...] = (y[0] * w[0][:, None] + y[1] * w[1][:, None]
                      + y[2] * w[2][:, None])

    return pl.pallas_call(
        body,
        grid=(SB // bsr,),
        in_specs=[
            pl.BlockSpec((K, bsr, H), lambda i: (0, i, 0)),
            pl.BlockSpec((K, bsr), lambda i: (0, i)),
        ],
        out_specs=pl.BlockSpec((bsr, H), lambda i: (i, 0)),
        out_shape=jax.ShapeDtypeStruct((SB, H), jnp.float32),
    )(yg3, w3)


def kernel(hidden_states, router_w, gate_w, up_w, down_w):
    x = hidden_states.reshape(SB, H)

    logits = x @ router_w.T
    probs = jax.nn.softmax(logits, axis=-1)
    p3, i3 = lax.top_k(probs, K)
    keep1 = p3[:, 0] <= TOP_P
    keep2 = (p3[:, 0] + p3[:, 1]) <= TOP_P
    w3 = jnp.stack([p3[:, 0],
                    jnp.where(keep1, p3[:, 1], 0.0),
                    jnp.where(keep2, p3[:, 2], 0.0)], axis=0)

    ef = i3.reshape(-1).astype(jnp.int32)
    cb = 128
    nc = A // cb
    oh = (ef.reshape(nc, cb)[:, :, None]
          == jnp.arange(E, dtype=jnp.int32)[None, None, :])
    ohf = oh.astype(jnp.float32)
    tri = (jnp.arange(cb)[:, None] >= jnp.arange(cb)[None, :]).astype(
        jnp.float32)
    intra = jnp.einsum('rc,ncE->nrE', tri, ohf,
                       preferred_element_type=jnp.float32)
    chunk_tot = intra[:, cb - 1, :]
    carry = jnp.cumsum(chunk_tot, axis=0) - chunk_tot
    counts = chunk_tot.sum(axis=0).astype(jnp.int32)
    rank = (intra + carry[:, None, :]).astype(jnp.int32).reshape(A, E)
    rank = jnp.take_along_axis(rank, ef[:, None], axis=1)[:, 0] - 1
    nb = (counts + BS_G - 1) // BS_G
    cnb = jnp.cumsum(nb)
    base = (cnb - nb) * BS_G
    slot = base[ef] + rank
    tok = jnp.zeros((NPAD,), jnp.int32).at[slot].set(
        jnp.arange(A, dtype=jnp.int32) // K)
    pos_sm = slot.reshape(SB, K).T.reshape(A)
    block_expert = jnp.minimum(
        jnp.searchsorted(cnb, jnp.arange(NBLK), side='right'),
        E - 1).astype(jnp.int32)

    xg = _sc_gather(x, tok, NPAD, 56)
    yg = _grouped_mlp(xg, gate_w, up_w, down_w, block_expert)
    yg3 = _sc_gather(yg, pos_sm, A, 48).reshape(K, SB, H)
    out = _combine(yg3, w3)
    return out.reshape(S, B, H)


# speedup vs baseline: 2.6229x; 1.8062x over previous
import functools

import jax
import jax.numpy as jnp
from jax import lax
from jax.experimental import pallas as pl
from jax.experimental.pallas import tpu as pltpu
from jax.experimental.pallas import tpu_sc as plsc

S, B, H, FF, E = 2048, 1, 1024, 2048, 8
TOP_P = 0.3
SB = S * B
K = 3
A = SB * K
BS_G = 256
NBLK = A // BS_G + E
NPAD = NBLK * BS_G
FB = 512
NW = 32


TB = 256


def _router(x, router_w):
    def body(x_ref, rw_ref, w3_ref, ef_ref, rank_ref, cnt_ref, carry_ref):
        i = pl.program_id(0)
        logit = lax.dot_general(x_ref[...], rw_ref[...],
                                (((1,), (1,)), ((), ())),
                                preferred_element_type=jnp.float32)
        m = jnp.max(logit, axis=1, keepdims=True)
        ex = jnp.exp(logit - m)
        p = ex / jnp.sum(ex, axis=1, keepdims=True)
        iota_e = lax.broadcasted_iota(jnp.int32, (TB, E), 1)

        def pick(pcur):
            mx = jnp.max(pcur, axis=1)
            am = jnp.min(jnp.where(pcur == mx[:, None], iota_e, E), axis=1)
            return mx, am.astype(jnp.int32), jnp.where(
                iota_e == am[:, None], -1.0, pcur)

        p0, a0, pn = pick(p)
        p1, a1, pn = pick(pn)
        p2, a2, _ = pick(pn)
        w3_ref[...] = jnp.stack(
            [p0, jnp.where(p0 <= TOP_P, p1, 0.0),
             jnp.where(p0 + p1 <= TOP_P, p2, 0.0)], axis=0)
        efb = jnp.stack([a0, a1, a2], axis=0)
        ef_ref[...] = efb

        iota8 = lax.broadcasted_iota(jnp.int32, (E, 1, 1), 0)
        oh = (efb[None, :, :] == iota8).astype(jnp.float32)
        colsum = jnp.sum(oh, axis=1)
        tri_s = (lax.broadcasted_iota(jnp.int32, (TB, TB), 0)
                 < lax.broadcasted_iota(jnp.int32, (TB, TB), 1)
                 ).astype(jnp.float32)
        ecs = lax.dot_general(colsum, tri_s, (((1,), (0,)), ((), ())),
                              preferred_element_type=jnp.float32)

        @pl.when(i == 0)
        def _():
            carry_ref[...] = jnp.zeros_like(carry_ref)

        carry = carry_ref[...]
        vals = ecs[:, None, :] + carry[:, :, None]
        rank_ref[...] = jnp.sum(
            jnp.where(efb[None, :, :] == iota8, vals, 0.0), axis=0
        ).astype(jnp.int32)
        tot = carry + jnp.sum(colsum, axis=1, keepdims=True)
        carry_ref[...] = tot
        cnt_ref[...] = tot

    return pl.pallas_call(
        body,
        grid=(SB // TB,),
        in_specs=[
            pl.BlockSpec((TB, H), lambda i: (i, 0)),
            pl.BlockSpec((E, H), lambda i: (0, 0)),
        ],
        out_specs=[
            pl.BlockSpec((K, TB), lambda i: (0, i)),
            pl.BlockSpec((K, TB), lambda i: (0, i)),
            pl.BlockSpec((K, TB), lambda i: (0, i)),
            pl.BlockSpec((E, 1), lambda i: (0, 0)),
        ],
        out_shape=[
            jax.ShapeDtypeStruct((K, SB), jnp.float32),
            jax.ShapeDtypeStruct((K, SB), jnp.int32),
            jax.ShapeDtypeStruct((K, SB), jnp.int32),
            jax.ShapeDtypeStruct((E, 1), jnp.float32),
        ],
        scratch_shapes=[pltpu.VMEM((E, 1), jnp.float32)],
        compiler_params=pltpu.CompilerParams(
            dimension_semantics=("arbitrary",),
        ),
    )(x, router_w)


def _route_finalize(counts, ef, rank):
    def body(cnt_ref, ef_ref, rank_ref, slot_ref, blke_ref):
        c = cnt_ref[...]
        nb = jnp.floor((c + (BS_G - 1)) / BS_G)
        tri_l = (lax.broadcasted_iota(jnp.int32, (E, E), 1)
                 <= lax.broadcasted_iota(jnp.int32, (E, E), 0)
                 ).astype(jnp.float32)
        cnb = lax.dot_general(tri_l, nb, (((1,), (0,)), ((), ())),
                              preferred_element_type=jnp.float32)
        base = (cnb - nb) * BS_G
        efb = ef_ref[...]
        iota8 = lax.broadcasted_iota(jnp.int32, (E, 1, 1), 0)
        basemap = jnp.sum(
            jnp.where(efb[None, :, :] == iota8, base[:, :, None], 0.0),
            axis=0)
        slot_ref[...] = rank_ref[...] + basemap.astype(jnp.int32)
        iota_b = lax.broadcasted_iota(jnp.int32, (E, 128), 1)
        blke = jnp.sum((iota_b >= cnb.astype(jnp.int32)).astype(jnp.int32),
                       axis=0, keepdims=True)
        blke_ref[...] = jnp.minimum(blke, E - 1)

    return pl.pallas_call(
        body,
        grid=(1,),
        in_specs=[
            pl.BlockSpec((E, 1), lambda i: (0, 0)),
            pl.BlockSpec((K, SB), lambda i: (0, 0)),
            pl.BlockSpec((K, SB), lambda i: (0, 0)),
        ],
        out_specs=[
            pl.BlockSpec((K, SB), lambda i: (0, 0)),
            pl.BlockSpec((1, 128), lambda i: (0, 0)),
        ],
        out_shape=[
            jax.ShapeDtypeStruct((K, SB), jnp.int32),
            jax.ShapeDtypeStruct((1, 128), jnp.int32),
        ],
    )(counts, ef, rank)


def _sc_dispatch(x, slot4, npad):
    per_w = SB // NW
    chunk = 32
    nch = per_w // chunk
    mesh = plsc.VectorSubcoreMesh(core_axis_name="c", subcore_axis_name="s")

    @functools.partial(
        pl.kernel,
        out_type=jax.ShapeDtypeStruct((npad, H), jnp.float32),
        mesh=mesh,
        scratch_types=[
            pltpu.VMEM((K, nch, chunk), jnp.int32),
            pltpu.VMEM((chunk, H), jnp.float32),
            pltpu.VMEM((chunk, H), jnp.float32),
            pltpu.SemaphoreType.DMA,
            pltpu.SemaphoreType.DMA,
        ],
    )
    def k(x_hbm, slot_hbm, out_hbm, idx_v, rows0, rows1, sem0, sem1):
        wid = lax.axis_index("s") * 2 + lax.axis_index("c")
        base = pl.multiple_of(wid * per_w, chunk)
        pltpu.sync_copy(slot_hbm.at[wid], idx_v)
        bufs = (rows0, rows1)
        sems = (sem0, sem1)
        reads = [
            pltpu.async_copy(
                x_hbm.at[pl.ds(base + ci * chunk, chunk)],
                bufs[ci % 2], sems[ci % 2])
            for ci in range(min(2, nch))
        ]
        for ci in range(nch):
            reads[ci].wait()
            for kk in range(K):
                pltpu.sync_copy(bufs[ci % 2], out_hbm.at[idx_v.at[kk, ci]])
            if ci + 2 < nch:
                reads.append(pltpu.async_copy(
                    x_hbm.at[pl.ds(base + (ci + 2) * chunk, chunk)],
                    bufs[ci % 2], sems[ci % 2]))

    return k(x, slot4)


def _sc_gather(table, idx, n_rows, chunk):
    per_w = n_rows // NW
    nch = per_w // chunk
    h = table.shape[1]
    mesh = plsc.VectorSubcoreMesh(core_axis_name="c", subcore_axis_name="s")

    @functools.partial(
        pl.kernel,
        out_type=jax.ShapeDtypeStruct((n_rows, h), jnp.float32),
        mesh=mesh,
        scratch_types=[
            pltpu.VMEM((per_w,), jnp.int32),
            pltpu.VMEM((chunk, h), jnp.float32),
            pltpu.VMEM((chunk, h), jnp.float32),
            pltpu.SemaphoreType.DMA,
            pltpu.SemaphoreType.DMA,
        ],
    )
    def k(table_hbm, idx_hbm, out_hbm, idx_v, rows0, rows1, sem0, sem1):
        wid = lax.axis_index("s") * 2 + lax.axis_index("c")
        base = pl.multiple_of(wid * per_w, chunk)
        pltpu.sync_copy(idx_hbm.at[pl.ds(base, per_w)], idx_v)
        bufs = (rows0, rows1)
        sems = (sem0, sem1)
        copies = [
            pltpu.async_copy(
                table_hbm.at[idx_v.at[pl.ds(ci * chunk, chunk)]],
                bufs[ci % 2], sems[ci % 2])
            for ci in range(min(2, nch))
        ]
        for ci in range(nch):
            copies[ci].wait()
            pltpu.sync_copy(
                bufs[ci % 2],
                out_hbm.at[pl.ds(pl.multiple_of(base + ci * chunk, 8), chunk)])
            if ci + 2 < nch:
                copies.append(pltpu.async_copy(
                    table_hbm.at[idx_v.at[pl.ds((ci + 2) * chunk, chunk)]],
                    bufs[ci % 2], sems[ci % 2]))

    return k(table, idx)


def _grouped_mlp(xg, gate_w, up_w, down_w, block_expert):
    grid_spec = pltpu.PrefetchScalarGridSpec(
        num_scalar_prefetch=1,
        grid=(NBLK,),
        in_specs=[
            pl.BlockSpec((BS_G, H), lambda b, be: (b, 0)),
            pl.BlockSpec((1, FF, H), lambda b, be: (be[b], 0, 0)),
            pl.BlockSpec((1, FF, H), lambda b, be: (be[b], 0, 0)),
            pl.BlockSpec((1, H, FF), lambda b, be: (be[b], 0, 0)),
        ],
        out_specs=pl.BlockSpec((BS_G, H), lambda b, be: (b, 0)),
    )

    def body(be_ref, x_ref, g_ref, u_ref, d_ref, o_ref):
        x = x_ref[...]
        g = lax.dot_general(x, g_ref[0], (((1,), (1,)), ((), ())),
                            preferred_element_type=jnp.float32)
        u = lax.dot_general(x, u_ref[0], (((1,), (1,)), ((), ())),
                            preferred_element_type=jnp.float32)
        hmid = (g * jax.nn.sigmoid(g)) * u
        o_ref[...] = lax.dot_general(hmid, d_ref[0], (((1,), (1,)), ((), ())),
                                     preferred_element_type=jnp.float32)

    return pl.pallas_call(
        body,
        grid_spec=grid_spec,
        out_shape=jax.ShapeDtypeStruct((NPAD, H), jnp.float32),
        compiler_params=pltpu.CompilerParams(
            dimension_semantics=("arbitrary",),
        ),
    )(block_expert, xg, gate_w, up_w, down_w)


def _combine(yg3, w3):
    bsr = 256

    def body(y_ref, w_ref, o_ref):
        y = y_ref[...]
        w = w_ref[...]
        o_ref[...] = (y[0] * w[0][:, None] + y[1] * w[1][:, None]
                      + y[2] * w[2][:, None])

    return pl.pallas_call(
        body,
        grid=(SB // bsr,),
        in_specs=[
            pl.BlockSpec((K, bsr, H), lambda i: (0, i, 0)),
            pl.BlockSpec((K, bsr), lambda i: (0, i)),
        ],
        out_specs=pl.BlockSpec((bsr, H), lambda i: (i, 0)),
        out_shape=jax.ShapeDtypeStruct((SB, H), jnp.float32),
    )(yg3, w3)


def kernel(hidden_states, router_w, gate_w, up_w, down_w):
    x = hidden_states.reshape(SB, H)

    w3, ef, rank, counts = _router(x, router_w)
    slot_sm, blke = _route_finalize(counts, ef, rank)
    pos_sm = slot_sm.reshape(A)
    chunk = 32
    slot4 = slot_sm.reshape(K, NW, (SB // NW) // chunk, chunk).transpose(
        1, 0, 2, 3)
    block_expert = blke[0, :NBLK]

    xg = _sc_dispatch(x, slot4, NPAD)
    yg = _grouped_mlp(xg, gate_w, up_w, down_w, block_expert)
    yg3 = _sc_gather(yg, pos_sm, A, 48).reshape(K, SB, H)
    out = _combine(yg3, w3)
    return out.reshape(S, B, H)
